# Initial kernel scaffold; baseline (speedup 1.0000x reference)
import jax
import jax.numpy as jnp
from jax.experimental import pallas as pl


def kernel(embeddings, positive_nodes, negative_nodes):
    raise NotImplementedError("write your pallas kernel here")


# speedup vs baseline: 2.3294x; 2.3294x over previous
import functools

import jax
import jax.numpy as jnp
from jax import lax
from jax.experimental import pallas as pl
from jax.experimental.pallas import tpu as pltpu
from jax.experimental.pallas import tpu_sc as plsc

_EPS = 1e-8

_NC, _NS, _L = 2, 16, 16
_NW = _NC * _NS
_N = 50000
_PER_W = 1568
_PAD_B = _NW * _PER_W
_CHUNK = 112
_NCHUNK = _PER_W // _CHUNK
_NGRP = _CHUNK // _L
_D = 128
_KREG = _D // _L


_GATHER_DNUMS = lax.GatherDimensionNumbers(
    offset_dims=(), collapsed_slice_dims=(0,), start_index_map=(0,))


def _lane_gather(x, idx):
    return lax.gather(x, idx.reshape(_L, 1), _GATHER_DNUMS, (1,),
                      mode=lax.GatherScatterMode.PROMISE_IN_BOUNDS)


def _hsum_splat(x):
    lanes = lax.iota(jnp.int32, _L)
    for d in (8, 4, 2, 1):
        x = x + _lane_gather(x, lanes ^ d)
    return x


def _bcast_lane(x, i):
    return _lane_gather(x, jnp.full((_L,), i, dtype=jnp.int32))


def _nr_rsqrt(s):
    s1 = s * 2.0**24
    y_scale = jnp.full((_L,), 2.0**12, dtype=jnp.float32)
    for e in (32, 16, 8, 4, 2, 1):
        big = s1 >= 2.0**e
        s1 = jnp.where(big, s1 * 2.0**-e, s1)
        y_scale = y_scale * jnp.where(big, jnp.float32(2.0 ** (-e / 2)),
                                      jnp.float32(1.0))
    y = 2.0 / (1.0 + s1)
    for _ in range(3):
        y = y * (1.5 - 0.5 * s1 * y * y)
    return y * y_scale


@functools.partial(
    pl.kernel,
    mesh=plsc.VectorSubcoreMesh(core_axis_name="c", subcore_axis_name="s"),
    out_type=jax.ShapeDtypeStruct((_NW, 4, _D), jnp.float32),
    scratch_types=[
        pltpu.VMEM((_PER_W,), jnp.int32),
        pltpu.VMEM((_PER_W,), jnp.int32),
        pltpu.VMEM((_CHUNK, _D), jnp.float32),
        pltpu.VMEM((_CHUNK, _D), jnp.float32),
        pltpu.VMEM((4, _D), jnp.float32),
        pltpu.SemaphoreType.DMA,
        pltpu.SemaphoreType.DMA,
    ],
)
def _partials(emb_hbm, pos_hbm, neg_hbm, out_hbm, idx_p, idx_n, buf0, buf1,
              out_v, sem0, sem1):
    wid = lax.axis_index("s") * _NC + lax.axis_index("c")
    base = wid * _PER_W
    pltpu.sync_copy(pos_hbm.at[pl.ds(base, _PER_W)], idx_p)
    pltpu.sync_copy(neg_hbm.at[pl.ds(base, _PER_W)], idx_n)

    def src(idx_v, cc):
        return emb_hbm.at[idx_v.at[pl.ds(cc * _CHUNK, _CHUNK)]]

    def fire(idx_v, cc, buf, sem):
        @pl.when(cc < _NCHUNK)
        def _():
            pltpu.async_copy(src(idx_v, jnp.minimum(cc, _NCHUNK - 1)),
                             buf, sem)

    def drain(idx_v, cc, buf, sem):
        pltpu.make_async_copy(src(idx_v, cc), buf, sem).wait()

    def process(buf, accs, cc):
        first_pad = _N - base - cc * _CHUNK

        @pl.when(first_pad < _CHUNK)
        def _():
            z = jnp.zeros((_L,), jnp.float32)

            def zbody(i, carry):
                for k in range(_KREG):
                    buf[i, pl.ds(k * _L, _L)] = z
                return carry

            lax.fori_loop(jnp.maximum(first_pad, 0), _CHUNK, zbody, 0)

        lanes = lax.iota(jnp.int32, _L)

        def gbody(g, accs):
            base_r = g * _L

            def p1(i, sqpack):
                v = [buf[base_r + i, pl.ds(k * _L, _L)]
                     for k in range(_KREG)]
                sq = v[0] * v[0]
                for k in range(1, _KREG):
                    sq = sq + v[k] * v[k]
                tot = _hsum_splat(sq)
                return jnp.where(lanes == i, tot, sqpack)

            sqpack = lax.fori_loop(0, _L, p1, jnp.zeros((_L,), jnp.float32))
            norm = sqpack * _nr_rsqrt(sqpack)
            w_grp = 1.0 / jnp.maximum(norm, _EPS)

            def p2(i, accs):
                wv = _bcast_lane(w_grp, i)
                v = [buf[base_r + i, pl.ds(k * _L, _L)]
                     for k in range(_KREG)]
                a_s = tuple(accs[k] + v[k] for k in range(_KREG))
                a_w = tuple(accs[_KREG + k] + wv * v[k]
                            for k in range(_KREG))
                return a_s + a_w

            return lax.fori_loop(0, _L, p2, accs)

        return lax.fori_loop(0, _NGRP, gbody, accs)

    for li, idx_v in enumerate((idx_p, idx_n)):
        fire(idx_v, 0, buf0, sem0)
        fire(idx_v, 1, buf1, sem1)

        def pipe_body(g, accs, idx_v=idx_v):
            c0 = 2 * g
            c1 = 2 * g + 1
            drain(idx_v, c0, buf0, sem0)
            accs = process(buf0, accs, c0)
            fire(idx_v, c0 + 2, buf0, sem0)
            drain(idx_v, c1, buf1, sem1)
            accs = process(buf1, accs, c1)
            fire(idx_v, c1 + 2, buf1, sem1)
            return accs

        accs = tuple(jnp.zeros((_L,), jnp.float32) for _ in range(2 * _KREG))
        accs = lax.fori_loop(0, _NCHUNK // 2, pipe_body, accs)
        for k in range(_KREG):
            out_v[2 * li + 0, pl.ds(k * _L, _L)] = accs[k]
            out_v[2 * li + 1, pl.ds(k * _L, _L)] = accs[_KREG + k]

    pltpu.sync_copy(out_v, out_hbm.at[wid])


def _side_loss(s_vec, w_vec):
    c = s_vec / _N
    cnorm = jnp.maximum(jnp.sqrt(jnp.sum(c * c)), _EPS)
    mean_cos = jnp.dot(w_vec, c) / (_N * cnorm)
    return 2.0 - 2.0 * mean_cos


def kernel(embeddings, positive_nodes, negative_nodes):
    pad = jnp.zeros((_PAD_B - _N,), jnp.int32)
    pos = jnp.concatenate([positive_nodes.astype(jnp.int32), pad])
    neg = jnp.concatenate([negative_nodes.astype(jnp.int32), pad])
    parts = _partials(embeddings, pos, neg)
    tot = jnp.sum(parts, axis=0)
    pos_loss = _side_loss(tot[0], tot[1])
    neg_loss = _side_loss(tot[2], tot[3])
    return (pos_loss + neg_loss) / 2.0


# speedup vs baseline: 2.3579x; 1.0123x over previous
import functools

import jax
import jax.numpy as jnp
from jax import lax
from jax.experimental import pallas as pl
from jax.experimental.pallas import tpu as pltpu
from jax.experimental.pallas import tpu_sc as plsc

_EPS = 1e-8

_NC, _NS, _L = 2, 16, 16
_NW = _NC * _NS
_N = 50000
_PER_W = 1568
_PAD_B = _NW * _PER_W
_CHUNK = 112
_NCHUNK = _PER_W // _CHUNK
_NGRP = _CHUNK // _L
_D = 128
_KREG = _D // _L


_GATHER_DNUMS = lax.GatherDimensionNumbers(
    offset_dims=(), collapsed_slice_dims=(0,), start_index_map=(0,))


def _lane_gather(x, idx):
    return lax.gather(x, idx.reshape(_L, 1), _GATHER_DNUMS, (1,),
                      mode=lax.GatherScatterMode.PROMISE_IN_BOUNDS)


def _hsum_splat(x):
    lanes = lax.iota(jnp.int32, _L)
    for d in (8, 4, 2, 1):
        x = x + _lane_gather(x, lanes ^ d)
    return x


def _bcast_lane(x, i):
    return _lane_gather(x, jnp.full((_L,), i, dtype=jnp.int32))


def _nr_rsqrt(s):
    s1 = s * 2.0**24
    y_scale = jnp.full((_L,), 2.0**12, dtype=jnp.float32)
    for e in (32, 16, 8, 4, 2, 1):
        big = s1 >= 2.0**e
        s1 = jnp.where(big, s1 * 2.0**-e, s1)
        y_scale = y_scale * jnp.where(big, jnp.float32(2.0 ** (-e / 2)),
                                      jnp.float32(1.0))
    y = 2.0 / (1.0 + s1)
    for _ in range(3):
        y = y * (1.5 - 0.5 * s1 * y * y)
    return y * y_scale


@functools.partial(
    pl.kernel,
    mesh=plsc.VectorSubcoreMesh(core_axis_name="c", subcore_axis_name="s"),
    out_type=jax.ShapeDtypeStruct((_NW, 4, _D), jnp.float32),
    scratch_types=[
        pltpu.VMEM((_PER_W,), jnp.int32),
        pltpu.VMEM((_PER_W,), jnp.int32),
        pltpu.VMEM((_CHUNK, _D), jnp.float32),
        pltpu.VMEM((_CHUNK, _D), jnp.float32),
        pltpu.VMEM((4, _D), jnp.float32),
        pltpu.SemaphoreType.DMA,
        pltpu.SemaphoreType.DMA,
    ],
)
def _partials(emb_hbm, pos_hbm, neg_hbm, out_hbm, idx_p, idx_n, buf0, buf1,
              out_v, sem0, sem1):
    wid = lax.axis_index("s") * _NC + lax.axis_index("c")
    base = wid * _PER_W
    pltpu.sync_copy(pos_hbm.at[pl.ds(base, _PER_W)], idx_p)
    pltpu.sync_copy(neg_hbm.at[pl.ds(base, _PER_W)], idx_n)

    def src(idx_v, cc):
        return emb_hbm.at[idx_v.at[pl.ds(cc * _CHUNK, _CHUNK)]]

    def fire(idx_v, cc, buf, sem):
        @pl.when(cc < _NCHUNK)
        def _():
            pltpu.async_copy(src(idx_v, jnp.minimum(cc, _NCHUNK - 1)),
                             buf, sem)

    def drain(idx_v, cc, buf, sem):
        pltpu.make_async_copy(src(idx_v, cc), buf, sem).wait()

    def process(buf, accs, cc):
        first_pad = _N - base - cc * _CHUNK

        @pl.when(first_pad < _CHUNK)
        def _():
            z = jnp.zeros((_L,), jnp.float32)

            def zbody(i, carry):
                for k in range(_KREG):
                    buf[i, pl.ds(k * _L, _L)] = z
                return carry

            lax.fori_loop(jnp.maximum(first_pad, 0), _CHUNK, zbody, 0)

        lanes = lax.iota(jnp.int32, _L)

        def gbody(g, accs):
            base_r = g * _L

            def p1(i, sqpack):
                v = [buf[base_r + i, pl.ds(k * _L, _L)]
                     for k in range(_KREG)]
                sq = v[0] * v[0]
                for k in range(1, _KREG):
                    sq = sq + v[k] * v[k]
                tot = _hsum_splat(sq)
                return jnp.where(lanes == i, tot, sqpack)

            sqpack = lax.fori_loop(0, _L, p1, jnp.zeros((_L,), jnp.float32),
                                   unroll=4)
            norm = sqpack * _nr_rsqrt(sqpack)
            w_grp = 1.0 / jnp.maximum(norm, _EPS)

            def p2(i, accs):
                wv = _bcast_lane(w_grp, i)
                v = [buf[base_r + i, pl.ds(k * _L, _L)]
                     for k in range(_KREG)]
                a_s = tuple(accs[k] + v[k] for k in range(_KREG))
                a_w = tuple(accs[_KREG + k] + wv * v[k]
                            for k in range(_KREG))
                return a_s + a_w

            return lax.fori_loop(0, _L, p2, accs, unroll=4)

        return lax.fori_loop(0, _NGRP, gbody, accs)

    for li, idx_v in enumerate((idx_p, idx_n)):
        fire(idx_v, 0, buf0, sem0)
        fire(idx_v, 1, buf1, sem1)

        def pipe_body(g, accs, idx_v=idx_v):
            c0 = 2 * g
            c1 = 2 * g + 1
            drain(idx_v, c0, buf0, sem0)
            accs = process(buf0, accs, c0)
            fire(idx_v, c0 + 2, buf0, sem0)
            drain(idx_v, c1, buf1, sem1)
            accs = process(buf1, accs, c1)
            fire(idx_v, c1 + 2, buf1, sem1)
            return accs

        accs = tuple(jnp.zeros((_L,), jnp.float32) for _ in range(2 * _KREG))
        accs = lax.fori_loop(0, _NCHUNK // 2, pipe_body, accs)
        for k in range(_KREG):
            out_v[2 * li + 0, pl.ds(k * _L, _L)] = accs[k]
            out_v[2 * li + 1, pl.ds(k * _L, _L)] = accs[_KREG + k]

    pltpu.sync_copy(out_v, out_hbm.at[wid])


def _side_loss(s_vec, w_vec):
    c = s_vec / _N
    cnorm = jnp.maximum(jnp.sqrt(jnp.sum(c * c)), _EPS)
    mean_cos = jnp.dot(w_vec, c) / (_N * cnorm)
    return 2.0 - 2.0 * mean_cos


def kernel(embeddings, positive_nodes, negative_nodes):
    pad = jnp.zeros((_PAD_B - _N,), jnp.int32)
    pos = jnp.concatenate([positive_nodes.astype(jnp.int32), pad])
    neg = jnp.concatenate([negative_nodes.astype(jnp.int32), pad])
    parts = _partials(embeddings, pos, neg)
    tot = jnp.sum(parts, axis=0)
    pos_loss = _side_loss(tot[0], tot[1])
    neg_loss = _side_loss(tot[2], tot[3])
    return (pos_loss + neg_loss) / 2.0
